# baseline (speedup 1.0000x reference)
import functools

import jax
import jax.numpy as jnp
import numpy as np
from jax.experimental import pallas as pl
from jax.experimental.pallas import tpu as pltpu
from jax.sharding import Mesh, PartitionSpec as P


def _stage1_kernel(x_ref, w1a_ref, b1a_ref, w1b_ref, b1b_ref, w2b_ref,
                   b2b_ref, ha_ref, pb_ref):
    x = x_ref[...]
    ha = jnp.dot(x, w1a_ref[...], preferred_element_type=jnp.float32)
    ha = jnp.maximum(ha + b1a_ref[...], 0.0)
    ha_ref[...] = ha.astype(ha_ref.dtype)

    hb = jnp.dot(x, w1b_ref[...], preferred_element_type=jnp.float32)
    hb = jnp.maximum(hb + b1b_ref[...], 0.0)
    lb = jnp.dot(hb.astype(x.dtype), w2b_ref[...],
                 preferred_element_type=jnp.float32) + b2b_ref[...]
    mb = jnp.max(lb, axis=1, keepdims=True)
    eb = jnp.exp(lb - mb)
    pb = eb / jnp.sum(eb, axis=1, keepdims=True)
    pb_ref[...] = pb.astype(pb_ref.dtype)


def _stage2_kernel(ha_ref, pb_ref, w2a_ref, b2a_ref, e_ref, out_ref,
                   p_ref, st_ref, *, nkj, nsub, nk, size):
    j = pl.program_id(1)
    ha = ha_ref[...]
    sck = w2a_ref.shape[1] // nsub
    for s in range(nsub):
        la = jnp.dot(ha, w2a_ref[:, s * sck:(s + 1) * sck],
                     preferred_element_type=jnp.float32)
        la = la + b2a_ref[:, s * sck:(s + 1) * sck]
        mc = jnp.max(la, axis=1, keepdims=True)
        p = jnp.exp(la - mc)
        sc = jnp.sum(p, axis=1, keepdims=True)
        c = j * nsub + s
        p_ref[c] = p.astype(p_ref.dtype)
        st_ref[c, :, 0:1] = mc
        st_ref[c, :, 1:2] = sc

    @pl.when(j == nkj - 1)
    def _finalize():
        bblk = out_ref.shape[0]
        ck = p_ref.shape[2]
        g = ck // size

        m = st_ref[0, :, 0:1]
        for c in range(1, nk):
            m = jnp.maximum(m, st_ref[c, :, 0:1])
        s = jnp.zeros((bblk, 1), jnp.float32)
        for c in range(nk):
            s = s + st_ref[c, :, 1:2] * jnp.exp(st_ref[c, :, 0:1] - m)

        pbv = pb_ref[...]
        acc = jnp.zeros((bblk, size), jnp.float32)
        for c in range(nk):
            mc_ = st_ref[c, :, 0:1]
            fc = jnp.exp(mc_ - m)
            gc = s * jnp.exp(jnp.minimum(m - mc_, 70.0))
            thr = (jnp.dot(pbv, e_ref[:, c * ck:(c + 1) * ck],
                           preferred_element_type=jnp.float32) * gc
                   ).astype(p_ref.dtype)
            mins = jnp.minimum(p_ref[c], thr)
            part = mins[:, 0:size]
            for xx in range(1, g):
                part = jnp.maximum(part, mins[:, xx * size:(xx + 1) * size])
            acc = jnp.maximum(acc, part.astype(jnp.float32) * fc)
        out_ref[...] = acc / s


def kernel(x, W1a, b1a, W2a, b2a, W1b, b1b, W2b, b2b):
    ndev = len(jax.devices())
    if ndev > 1 and x.shape[0] % (512 * ndev) == 0:
        mesh = Mesh(np.array(jax.devices()), ("b",))
        m2 = P(None, None)
        v1 = P(None)
        fn = jax.shard_map(
            _kernel_impl, mesh=mesh,
            in_specs=(P("b", None), m2, v1, m2, v1, m2, v1, m2, v1),
            out_specs=P("b", None),
            check_vma=False,
        )
        return fn(x, W1a, b1a, W2a, b2a, W1b, b1b, W2b, b2b)
    return _kernel_impl(x, W1a, b1a, W2a, b2a, W1b, b1b, W2b, b2b)


def _kernel_impl(x, W1a, b1a, W2a, b2a, W1b, b1b, W2b, b2b):
    B = x.shape[0]
    H = W1a.shape[1]
    size = W2b.shape[1]
    S2 = W2a.shape[1]

    bblk = 512
    nb = B // bblk
    ck = 2048
    nk = S2 // ck
    nsub = 2
    ckj = ck * nsub
    nkj = S2 // ckj

    cdt = jnp.bfloat16
    xc = x.astype(cdt)
    w1a = W1a.astype(cdt)
    w1b = W1b.astype(cdt)
    w2b = W2b.astype(cdt)
    w2a = W2a.astype(cdt)
    ecols = jnp.arange(S2, dtype=jnp.int32) // size
    emat = (ecols[None, :] == jnp.arange(size, dtype=jnp.int32)[:, None]
            ).astype(cdt)

    ha, pb = pl.pallas_call(
        _stage1_kernel,
        grid=(nb,),
        in_specs=[
            pl.BlockSpec((bblk, x.shape[1]), lambda i: (i, 0)),
            pl.BlockSpec((x.shape[1], H), lambda i: (0, 0)),
            pl.BlockSpec((1, H), lambda i: (0, 0)),
            pl.BlockSpec((x.shape[1], H), lambda i: (0, 0)),
            pl.BlockSpec((1, H), lambda i: (0, 0)),
            pl.BlockSpec((H, size), lambda i: (0, 0)),
            pl.BlockSpec((1, size), lambda i: (0, 0)),
        ],
        out_specs=[
            pl.BlockSpec((bblk, H), lambda i: (i, 0)),
            pl.BlockSpec((bblk, size), lambda i: (i, 0)),
        ],
        out_shape=[
            jax.ShapeDtypeStruct((B, H), cdt),
            jax.ShapeDtypeStruct((B, size), cdt),
        ],
        compiler_params=pltpu.CompilerParams(
            dimension_semantics=("arbitrary",)),
    )(xc, w1a, b1a.reshape(1, H), w1b, b1b.reshape(1, H), w2b,
      b2b.reshape(1, size))

    out = pl.pallas_call(
        functools.partial(_stage2_kernel, nkj=nkj, nsub=nsub, nk=nk,
                          size=size),
        grid=(nb, nkj),
        in_specs=[
            pl.BlockSpec((bblk, H), lambda i, j: (i, 0)),
            pl.BlockSpec((bblk, size), lambda i, j: (i, 0)),
            pl.BlockSpec((H, ckj), lambda i, j: (0, j)),
            pl.BlockSpec((1, ckj), lambda i, j: (0, j)),
            pl.BlockSpec((size, S2), lambda i, j: (0, 0)),
        ],
        out_specs=pl.BlockSpec((bblk, size), lambda i, j: (i, 0)),
        out_shape=jax.ShapeDtypeStruct((B, size), jnp.float32),
        scratch_shapes=[
            pltpu.VMEM((nk, bblk, ck), jnp.bfloat16),
            pltpu.VMEM((nk, bblk, 128), jnp.float32),
        ],
        compiler_params=pltpu.CompilerParams(
            dimension_semantics=("arbitrary", "arbitrary")),
    )(ha, pb, w2a, b2a.reshape(1, S2), emat)
    return out


# speedup vs baseline: 4.9448x; 4.9448x over previous
import functools

import jax
import jax.numpy as jnp
import numpy as np
from jax.experimental import pallas as pl
from jax.experimental.pallas import tpu as pltpu
from jax.sharding import Mesh, PartitionSpec as P


def _stage1_kernel(x_ref, w1a_ref, b1a_ref, w1b_ref, b1b_ref, w2b_ref,
                   b2b_ref, ha_ref, pb_ref):
    x = x_ref[...]
    ha = jnp.dot(x, w1a_ref[...], preferred_element_type=jnp.float32)
    ha = jnp.maximum(ha + b1a_ref[...], 0.0)
    ha_ref[...] = ha.astype(ha_ref.dtype)

    hb = jnp.dot(x, w1b_ref[...], preferred_element_type=jnp.float32)
    hb = jnp.maximum(hb + b1b_ref[...], 0.0)
    lb = jnp.dot(hb.astype(x.dtype), w2b_ref[...],
                 preferred_element_type=jnp.float32) + b2b_ref[...]
    mb = jnp.max(lb, axis=1, keepdims=True)
    eb = jnp.exp(lb - mb)
    pb = eb / jnp.sum(eb, axis=1, keepdims=True)
    pb_ref[...] = pb.astype(pb_ref.dtype)


def _stage2_kernel(ha_ref, pb_ref, w2a_ref, b2a_ref, out_ref,
                   p_ref, st_ref, *, nkj, nsub, nk, size):
    j = pl.program_id(1)
    ha = ha_ref[...]
    sck = w2a_ref.shape[1] // nsub
    for s in range(nsub):
        la = jnp.dot(ha, w2a_ref[:, s * sck:(s + 1) * sck],
                     preferred_element_type=jnp.float32)
        la = la + b2a_ref[:, s * sck:(s + 1) * sck]
        mc = jnp.max(la, axis=1, keepdims=True)
        p = jnp.exp(la - mc)
        sc = jnp.sum(p, axis=1, keepdims=True)
        c = j * nsub + s
        p_ref[c] = p.astype(p_ref.dtype)
        st_ref[c, :, 0:1] = mc
        st_ref[c, :, 1:2] = sc

    @pl.when(j == nkj - 1)
    def _finalize():
        bblk = out_ref.shape[0]
        ck = p_ref.shape[2]
        g = ck // size

        m = st_ref[0, :, 0:1]
        for c in range(1, nk):
            m = jnp.maximum(m, st_ref[c, :, 0:1])
        s = jnp.zeros((bblk, 1), jnp.float32)
        for c in range(nk):
            s = s + st_ref[c, :, 1:2] * jnp.exp(st_ref[c, :, 0:1] - m)

        pbv = pb_ref[...].astype(jnp.float32)
        acc = jnp.zeros((bblk, size), jnp.float32)
        for c in range(nk):
            mc_ = st_ref[c, :, 0:1]
            fc = jnp.exp(mc_ - m)
            gc = s * jnp.exp(jnp.minimum(m - mc_, 70.0))
            thrc = (pbv[:, c * g:(c + 1) * g] * gc).astype(p_ref.dtype)
            part = None
            for xx in range(g):
                tile = p_ref[c, :, xx * size:(xx + 1) * size]
                t = jnp.minimum(tile, thrc[:, xx:xx + 1])
                part = t if part is None else jnp.maximum(part, t)
            acc = jnp.maximum(acc, part.astype(jnp.float32) * fc)
        out_ref[...] = acc / s


def kernel(x, W1a, b1a, W2a, b2a, W1b, b1b, W2b, b2b):
    return _kernel_impl(x, W1a, b1a, W2a, b2a, W1b, b1b, W2b, b2b)


def _kernel_impl(x, W1a, b1a, W2a, b2a, W1b, b1b, W2b, b2b):
    B = x.shape[0]
    H = W1a.shape[1]
    size = W2b.shape[1]
    S2 = W2a.shape[1]

    bblk = 512
    nb = B // bblk
    ck = 2048
    nk = S2 // ck
    nsub = 1
    ckj = ck * nsub
    nkj = S2 // ckj

    cdt = jnp.bfloat16
    xc = x.astype(cdt)
    w1a = W1a.astype(cdt)
    w1b = W1b.astype(cdt)
    w2b = W2b.astype(cdt)
    w2a = W2a.astype(cdt)

    ha, pb = pl.pallas_call(
        _stage1_kernel,
        grid=(nb,),
        in_specs=[
            pl.BlockSpec((bblk, x.shape[1]), lambda i: (i, 0)),
            pl.BlockSpec((x.shape[1], H), lambda i: (0, 0)),
            pl.BlockSpec((1, H), lambda i: (0, 0)),
            pl.BlockSpec((x.shape[1], H), lambda i: (0, 0)),
            pl.BlockSpec((1, H), lambda i: (0, 0)),
            pl.BlockSpec((H, size), lambda i: (0, 0)),
            pl.BlockSpec((1, size), lambda i: (0, 0)),
        ],
        out_specs=[
            pl.BlockSpec((bblk, H), lambda i: (i, 0)),
            pl.BlockSpec((bblk, size), lambda i: (i, 0)),
        ],
        out_shape=[
            jax.ShapeDtypeStruct((B, H), cdt),
            jax.ShapeDtypeStruct((B, size), cdt),
        ],
        compiler_params=pltpu.CompilerParams(
            dimension_semantics=("arbitrary",)),
    )(xc, w1a, b1a.reshape(1, H), w1b, b1b.reshape(1, H), w2b,
      b2b.reshape(1, size))

    out = pl.pallas_call(
        functools.partial(_stage2_kernel, nkj=nkj, nsub=nsub, nk=nk,
                          size=size),
        grid=(nb, nkj),
        in_specs=[
            pl.BlockSpec((bblk, H), lambda i, j: (i, 0)),
            pl.BlockSpec((bblk, size), lambda i, j: (i, 0)),
            pl.BlockSpec((H, ckj), lambda i, j: (0, j)),
            pl.BlockSpec((1, ckj), lambda i, j: (0, j)),
        ],
        out_specs=pl.BlockSpec((bblk, size), lambda i, j: (i, 0)),
        out_shape=jax.ShapeDtypeStruct((B, size), jnp.float32),
        scratch_shapes=[
            pltpu.VMEM((nk, bblk, ck), jnp.bfloat16),
            pltpu.VMEM((nk, bblk, 128), jnp.float32),
        ],
        compiler_params=pltpu.CompilerParams(
            dimension_semantics=("arbitrary", "arbitrary")),
    )(ha, pb, w2a, b2a.reshape(1, S2))
    return out


# speedup vs baseline: 5.0500x; 1.0213x over previous
import functools

import jax
import jax.numpy as jnp
import numpy as np
from jax.experimental import pallas as pl
from jax.experimental.pallas import tpu as pltpu
from jax.sharding import Mesh, PartitionSpec as P


def _stage1_kernel(x_ref, w1a_ref, b1a_ref, w1b_ref, b1b_ref, w2b_ref,
                   b2b_ref, ha_ref, pb_ref):
    x = x_ref[...]
    ha = jnp.dot(x, w1a_ref[...], preferred_element_type=jnp.float32)
    ha = jnp.maximum(ha + b1a_ref[...], 0.0)
    ha_ref[...] = ha.astype(ha_ref.dtype)

    hb = jnp.dot(x, w1b_ref[...], preferred_element_type=jnp.float32)
    hb = jnp.maximum(hb + b1b_ref[...], 0.0)
    lb = jnp.dot(hb.astype(x.dtype), w2b_ref[...],
                 preferred_element_type=jnp.float32) + b2b_ref[...]
    mb = jnp.max(lb, axis=1, keepdims=True)
    eb = jnp.exp(lb - mb)
    pb = eb / jnp.sum(eb, axis=1, keepdims=True)
    pb_ref[...] = pb.astype(pb_ref.dtype)


def _stage2_kernel(ha_ref, pb_ref, w2a_ref, b2a_ref, e_ref, out_ref,
                   p_ref, st_ref, *, nkj, nsub, nk, size):
    j = pl.program_id(1)
    ha = ha_ref[...]
    sck = w2a_ref.shape[1] // nsub
    las = []
    for s in range(nsub):
        la = jnp.dot(ha, w2a_ref[:, s * sck:(s + 1) * sck],
                     preferred_element_type=jnp.float32)
        las.append(la + b2a_ref[:, s * sck:(s + 1) * sck])
    for s in range(nsub):
        la = las[s]
        mc = jnp.max(la, axis=1, keepdims=True)
        p = jnp.exp(la - mc)
        sc = jnp.sum(p, axis=1, keepdims=True)
        c = j * nsub + s
        p_ref[c] = p.astype(p_ref.dtype)
        st_ref[c, :, 0:1] = mc
        st_ref[c, :, 1:2] = sc

    @pl.when(j == nkj - 1)
    def _finalize():
        bblk = out_ref.shape[0]
        ck = p_ref.shape[2]
        g = ck // size

        m = st_ref[0, :, 0:1]
        for c in range(1, nk):
            m = jnp.maximum(m, st_ref[c, :, 0:1])
        s = jnp.zeros((bblk, 1), jnp.float32)
        for c in range(nk):
            s = s + st_ref[c, :, 1:2] * jnp.exp(st_ref[c, :, 0:1] - m)

        pbv = pb_ref[...]
        acc = jnp.zeros((bblk, size), jnp.float32)
        for c in range(nk):
            mc_ = st_ref[c, :, 0:1]
            fc = jnp.exp(mc_ - m)
            gc = s * jnp.exp(jnp.minimum(m - mc_, 70.0))
            thr = (jnp.dot(pbv, e_ref[:, c * ck:(c + 1) * ck],
                           preferred_element_type=jnp.float32) * gc
                   ).astype(p_ref.dtype)
            mins = jnp.minimum(p_ref[c], thr)
            part = mins[:, 0:size]
            for xx in range(1, g):
                part = jnp.maximum(part, mins[:, xx * size:(xx + 1) * size])
            acc = jnp.maximum(acc, part.astype(jnp.float32) * fc)
        out_ref[...] = acc / s


def kernel(x, W1a, b1a, W2a, b2a, W1b, b1b, W2b, b2b):
    return _kernel_impl(x, W1a, b1a, W2a, b2a, W1b, b1b, W2b, b2b)


def _kernel_impl(x, W1a, b1a, W2a, b2a, W1b, b1b, W2b, b2b):
    B = x.shape[0]
    H = W1a.shape[1]
    size = W2b.shape[1]
    S2 = W2a.shape[1]

    bblk = 512
    nb = B // bblk
    ck = 2048
    nk = S2 // ck
    nsub = 2
    ckj = ck * nsub
    nkj = S2 // ckj

    cdt = jnp.bfloat16
    xc = x.astype(cdt)
    w1a = W1a.astype(cdt)
    w1b = W1b.astype(cdt)
    w2b = W2b.astype(cdt)
    w2a = W2a.astype(cdt)
    ecols = jnp.arange(S2, dtype=jnp.int32) // size
    emat = (ecols[None, :] == jnp.arange(size, dtype=jnp.int32)[:, None]
            ).astype(cdt)

    ha, pb = pl.pallas_call(
        _stage1_kernel,
        grid=(nb,),
        in_specs=[
            pl.BlockSpec((bblk, x.shape[1]), lambda i: (i, 0)),
            pl.BlockSpec((x.shape[1], H), lambda i: (0, 0)),
            pl.BlockSpec((1, H), lambda i: (0, 0)),
            pl.BlockSpec((x.shape[1], H), lambda i: (0, 0)),
            pl.BlockSpec((1, H), lambda i: (0, 0)),
            pl.BlockSpec((H, size), lambda i: (0, 0)),
            pl.BlockSpec((1, size), lambda i: (0, 0)),
        ],
        out_specs=[
            pl.BlockSpec((bblk, H), lambda i: (i, 0)),
            pl.BlockSpec((bblk, size), lambda i: (i, 0)),
        ],
        out_shape=[
            jax.ShapeDtypeStruct((B, H), cdt),
            jax.ShapeDtypeStruct((B, size), cdt),
        ],
        compiler_params=pltpu.CompilerParams(
            dimension_semantics=("arbitrary",)),
    )(xc, w1a, b1a.reshape(1, H), w1b, b1b.reshape(1, H), w2b,
      b2b.reshape(1, size))

    out = pl.pallas_call(
        functools.partial(_stage2_kernel, nkj=nkj, nsub=nsub, nk=nk,
                          size=size),
        grid=(nb, nkj),
        in_specs=[
            pl.BlockSpec((bblk, H), lambda i, j: (i, 0)),
            pl.BlockSpec((bblk, size), lambda i, j: (i, 0)),
            pl.BlockSpec((H, ckj), lambda i, j: (0, j)),
            pl.BlockSpec((1, ckj), lambda i, j: (0, j)),
            pl.BlockSpec((size, S2), lambda i, j: (0, 0)),
        ],
        out_specs=pl.BlockSpec((bblk, size), lambda i, j: (i, 0)),
        out_shape=jax.ShapeDtypeStruct((B, size), jnp.float32),
        scratch_shapes=[
            pltpu.VMEM((nk, bblk, ck), jnp.bfloat16),
            pltpu.VMEM((nk, bblk, 128), jnp.float32),
        ],
        compiler_params=pltpu.CompilerParams(
            dimension_semantics=("arbitrary", "arbitrary")),
    )(ha, pb, w2a, b2a.reshape(1, S2), emat)
    return out


# speedup vs baseline: 5.0585x; 1.0017x over previous
import functools

import jax
import jax.numpy as jnp
from jax.experimental import pallas as pl
from jax.experimental.pallas import tpu as pltpu


def _stage1_kernel(x_ref, w1a_ref, b1a_ref, w1b_ref, b1b_ref, w2b_ref,
                   b2b_ref, ha_ref, pb_ref):
    x = x_ref[...]
    ha = jnp.dot(x, w1a_ref[...], preferred_element_type=jnp.float32)
    ha = jnp.maximum(ha + b1a_ref[...], 0.0)
    ha_ref[...] = ha.astype(ha_ref.dtype)

    hb = jnp.dot(x, w1b_ref[...], preferred_element_type=jnp.float32)
    hb = jnp.maximum(hb + b1b_ref[...], 0.0)
    lb = jnp.dot(hb.astype(x.dtype), w2b_ref[...],
                 preferred_element_type=jnp.float32) + b2b_ref[...]
    mb = jnp.max(lb, axis=1, keepdims=True)
    eb = jnp.exp(lb - mb)
    pb = eb / jnp.sum(eb, axis=1, keepdims=True)
    pb_ref[...] = pb.astype(pb_ref.dtype)


def _stage2_kernel(ha_ref, pb_ref, w2a_ref, b2a_ref, e_ref, out_ref,
                   p_ref, st_ref, *, nkj, nsub, nk, size):
    j = pl.program_id(1)
    ha = ha_ref[...]
    sck = w2a_ref.shape[1] // nsub
    las = []
    for s in range(nsub):
        la = jnp.dot(ha, w2a_ref[:, s * sck:(s + 1) * sck],
                     preferred_element_type=jnp.float32)
        las.append(la + b2a_ref[:, s * sck:(s + 1) * sck])
    for s in range(nsub):
        la = las[s]
        mc = jnp.max(la, axis=1, keepdims=True)
        p = jnp.exp(la - mc)
        sc = jnp.sum(p, axis=1, keepdims=True)
        c = j * nsub + s
        p_ref[c] = p.astype(p_ref.dtype)
        st_ref[c, :, 0:1] = mc
        st_ref[c, :, 1:2] = sc

    @pl.when(j == nkj - 1)
    def _finalize():
        bblk = out_ref.shape[0]
        ck = p_ref.shape[2]
        g = ck // size

        m = st_ref[0, :, 0:1]
        for c in range(1, nk):
            m = jnp.maximum(m, st_ref[c, :, 0:1])
        s = jnp.zeros((bblk, 1), jnp.float32)
        for c in range(nk):
            s = s + st_ref[c, :, 1:2] * jnp.exp(st_ref[c, :, 0:1] - m)

        pbv = pb_ref[...]
        acc = jnp.zeros((bblk, size), jnp.float32)
        for c in range(nk):
            mc_ = st_ref[c, :, 0:1]
            fc = jnp.exp(mc_ - m)
            gc = s * jnp.exp(jnp.minimum(m - mc_, 70.0))
            thr = (jnp.dot(pbv, e_ref[:, c * ck:(c + 1) * ck],
                           preferred_element_type=jnp.float32) * gc
                   ).astype(p_ref.dtype)
            mins = jnp.minimum(p_ref[c], thr)
            part = mins[:, 0:size]
            for xx in range(1, g):
                part = jnp.maximum(part, mins[:, xx * size:(xx + 1) * size])
            acc = jnp.maximum(acc, part.astype(jnp.float32) * fc)
        out_ref[...] = acc / s


def kernel(x, W1a, b1a, W2a, b2a, W1b, b1b, W2b, b2b):
    return _kernel_impl(x, W1a, b1a, W2a, b2a, W1b, b1b, W2b, b2b)


def _kernel_impl(x, W1a, b1a, W2a, b2a, W1b, b1b, W2b, b2b):
    B = x.shape[0]
    H = W1a.shape[1]
    size = W2b.shape[1]
    S2 = W2a.shape[1]

    bblk = 512
    nb = B // bblk
    ck = 2048
    nk = S2 // ck
    nsub = 2
    ckj = ck * nsub
    nkj = S2 // ckj

    cdt = jnp.bfloat16
    xc = x.astype(cdt)
    w1a = W1a.astype(cdt)
    w1b = W1b.astype(cdt)
    w2b = W2b.astype(cdt)
    w2a = W2a.astype(cdt)
    ecols = jnp.arange(S2, dtype=jnp.int32) // size
    emat = (ecols[None, :] == jnp.arange(size, dtype=jnp.int32)[:, None]
            ).astype(cdt)

    ha, pb = pl.pallas_call(
        _stage1_kernel,
        grid=(nb,),
        in_specs=[
            pl.BlockSpec((bblk, x.shape[1]), lambda i: (i, 0)),
            pl.BlockSpec((x.shape[1], H), lambda i: (0, 0)),
            pl.BlockSpec((1, H), lambda i: (0, 0)),
            pl.BlockSpec((x.shape[1], H), lambda i: (0, 0)),
            pl.BlockSpec((1, H), lambda i: (0, 0)),
            pl.BlockSpec((H, size), lambda i: (0, 0)),
            pl.BlockSpec((1, size), lambda i: (0, 0)),
        ],
        out_specs=[
            pl.BlockSpec((bblk, H), lambda i: (i, 0)),
            pl.BlockSpec((bblk, size), lambda i: (i, 0)),
        ],
        out_shape=[
            jax.ShapeDtypeStruct((B, H), cdt),
            jax.ShapeDtypeStruct((B, size), cdt),
        ],
        compiler_params=pltpu.CompilerParams(
            dimension_semantics=("arbitrary",)),
    )(xc, w1a, b1a.reshape(1, H), w1b, b1b.reshape(1, H), w2b,
      b2b.reshape(1, size))

    out = pl.pallas_call(
        functools.partial(_stage2_kernel, nkj=nkj, nsub=nsub, nk=nk,
                          size=size),
        grid=(nb, nkj),
        in_specs=[
            pl.BlockSpec((bblk, H), lambda i, j: (i, 0)),
            pl.BlockSpec((bblk, size), lambda i, j: (i, 0)),
            pl.BlockSpec((H, ckj), lambda i, j: (0, j)),
            pl.BlockSpec((1, ckj), lambda i, j: (0, j)),
            pl.BlockSpec((size, S2), lambda i, j: (0, 0)),
        ],
        out_specs=pl.BlockSpec((bblk, size), lambda i, j: (i, 0)),
        out_shape=jax.ShapeDtypeStruct((B, size), jnp.float32),
        scratch_shapes=[
            pltpu.VMEM((nk, bblk, ck), jnp.bfloat16),
            pltpu.VMEM((nk, bblk, 128), jnp.float32),
        ],
        compiler_params=pltpu.CompilerParams(
            dimension_semantics=("arbitrary", "arbitrary")),
    )(ha, pb, w2a, b2a.reshape(1, S2), emat)
    return out
